# baseline (speedup 1.0000x reference)
import functools

import jax
import jax.numpy as jnp
from jax.experimental import pallas as pl
from jax.experimental.pallas import tpu as pltpu

_B = 16384
_D = 64
_K = 1000
_KP = 1024
_BS = 1024
_NB = _B // _BS
_HI = 32

_INV_SQRT_2PI = 0.3989422804014327

_CONTRACT0 = (((0,), (0,)), ((), ()))


def _cluster_kernel(feature_ref, cnt_ref, label_ref, weight_ref,
                    cosf_ref, meang_ref):
    eps = 1e-8
    cnt = cnt_ref[...]
    cnorm = jnp.sqrt(jnp.sum(cnt * cnt, axis=0, keepdims=True))
    cnn = cnt / jnp.maximum(cnorm, eps)

    col_iota = jax.lax.broadcasted_iota(jnp.int32, (_BS, _KP), 1)
    valid = col_iota < _K
    iota32 = jax.lax.broadcasted_iota(jnp.int32, (_BS, _HI), 1)

    def onehots(lab):
        oh_hi = (iota32 == (lab >> 5)).astype(jnp.float32)
        oh_lo = (iota32 == (lab & 31)).astype(jnp.float32)
        return oh_hi, oh_lo

    def phase1(j, carry):
        counts, sums = carry
        f = feature_ref[pl.ds(j * _BS, _BS), :]
        fnorm = jnp.sqrt(jnp.sum(f * f, axis=1, keepdims=True))
        fn = f / jnp.maximum(fnorm, eps)
        cos = jnp.dot(fn, cnn, preferred_element_type=jnp.float32)
        cos = jnp.where(valid, cos, -2.0)
        cos_f = jnp.max(cos, axis=1, keepdims=True)
        lab = jnp.min(jnp.where(cos == cos_f, col_iota, _KP), axis=1,
                      keepdims=True)
        oh_hi, oh_lo = onehots(lab)
        counts = counts + jax.lax.dot_general(
            oh_hi, oh_lo, _CONTRACT0, preferred_element_type=jnp.float32, precision=jax.lax.Precision.HIGHEST)
        sums = sums + jax.lax.dot_general(
            oh_hi * cos_f, oh_lo, _CONTRACT0,
            preferred_element_type=jnp.float32, precision=jax.lax.Precision.HIGHEST)
        cosf_ref[pl.ds(j * _BS, _BS), :] = cos_f
        label_ref[pl.ds(j * _BS, _BS), :] = lab.astype(jnp.float32)
        return counts, sums

    zero_mat = jnp.zeros((_HI, _HI), jnp.float32)
    counts, sums = jax.lax.fori_loop(0, _NB, phase1, (zero_mat, zero_mat))
    mean = sums / jnp.maximum(counts, 1.0)

    def phase2(j, sq):
        cos_f = cosf_ref[pl.ds(j * _BS, _BS), :]
        lab = label_ref[pl.ds(j * _BS, _BS), :].astype(jnp.int32)
        oh_hi, oh_lo = onehots(lab)
        rows = jnp.dot(oh_hi, mean, preferred_element_type=jnp.float32, precision=jax.lax.Precision.HIGHEST)
        mean_g = jnp.sum(rows * oh_lo, axis=1, keepdims=True)
        meang_ref[pl.ds(j * _BS, _BS), :] = mean_g
        d2 = (cos_f - mean_g) ** 2
        return sq + jax.lax.dot_general(
            oh_hi * d2, oh_lo, _CONTRACT0, preferred_element_type=jnp.float32, precision=jax.lax.Precision.HIGHEST)

    sq = jax.lax.fori_loop(0, _NB, phase2, zero_mat)
    var = sq / jnp.maximum(counts - 1.0, 1.0)
    inv_std = jax.lax.rsqrt(jnp.maximum(var, 1e-12))

    def phase3(j, _):
        cos_f = cosf_ref[pl.ds(j * _BS, _BS), :]
        lab = label_ref[pl.ds(j * _BS, _BS), :].astype(jnp.int32)
        oh_hi, oh_lo = onehots(lab)
        mean_g = meang_ref[pl.ds(j * _BS, _BS), :]
        rows = jnp.dot(oh_hi, inv_std, preferred_element_type=jnp.float32, precision=jax.lax.Precision.HIGHEST)
        isd_g = jnp.sum(rows * oh_lo, axis=1, keepdims=True)
        z = (cos_f - mean_g) * isd_g
        pdf = jnp.exp(-0.5 * z * z) * isd_g * _INV_SQRT_2PI
        w = jnp.where(cos_f < mean_g, pdf, 1.0)
        weight_ref[pl.ds(j * _BS, _BS), :] = w
        return 0

    jax.lax.fori_loop(0, _NB, phase3, 0)


@functools.partial(jax.jit, static_argnames=())
def kernel(feature, pred, unlabeled_index, centroids):
    del pred, unlabeled_index
    label2d, weight2d = pl.pallas_call(
        _cluster_kernel,
        out_shape=(
            jax.ShapeDtypeStruct((_B, 1), jnp.float32),
            jax.ShapeDtypeStruct((_B, 1), jnp.float32),
        ),
        scratch_shapes=[
            pltpu.VMEM((_B, 1), jnp.float32),
            pltpu.VMEM((_B, 1), jnp.float32),
        ],
    )(feature, jnp.zeros((_D, _KP), jnp.float32).at[:, :_K].set(centroids.T))
    return label2d.reshape(_B), weight2d.reshape(_B)


# speedup vs baseline: 1.0356x; 1.0356x over previous
import functools

import jax
import jax.numpy as jnp
from jax.experimental import pallas as pl
from jax.experimental.pallas import tpu as pltpu

_B = 16384
_D = 64
_K = 1000
_KP = 1024
_BS = 1024
_NB = _B // _BS
_HI = 32

_INV_SQRT_2PI = 0.3989422804014327

_CONTRACT0 = (((0,), (0,)), ((), ()))
_PREC = jax.lax.Precision.HIGHEST


def _cluster_kernel(feature_ref, cnt_ref, label_ref, weight_ref, cosf_ref):
    eps = 1e-8
    cnt = cnt_ref[...]
    cnorm = jnp.sqrt(jnp.sum(cnt * cnt, axis=0, keepdims=True))
    cnn = cnt / jnp.maximum(cnorm, eps)

    col_iota = jax.lax.broadcasted_iota(jnp.int32, (_BS, _KP), 1)
    valid = col_iota < _K
    iota32 = jax.lax.broadcasted_iota(jnp.int32, (_BS, _HI), 1)

    def onehots(lab):
        oh_hi = (iota32 == (lab >> 5)).astype(jnp.float32)
        oh_lo = (iota32 == (lab & 31)).astype(jnp.float32)
        return oh_hi, oh_lo

    def phase1(j, carry):
        counts, sums, sumsq = carry
        f = feature_ref[pl.ds(j * _BS, _BS), :]
        fnorm = jnp.sqrt(jnp.sum(f * f, axis=1, keepdims=True))
        fn = f / jnp.maximum(fnorm, eps)
        cos = jnp.dot(fn, cnn, preferred_element_type=jnp.float32)
        cos = jnp.where(valid, cos, -2.0)
        cos_f = jnp.max(cos, axis=1, keepdims=True)
        lab = jnp.min(jnp.where(cos == cos_f, col_iota, _KP), axis=1,
                      keepdims=True)
        oh_hi, oh_lo = onehots(lab)
        ohv = oh_hi * cos_f
        counts = counts + jax.lax.dot_general(
            oh_hi, oh_lo, _CONTRACT0, preferred_element_type=jnp.float32,
            precision=_PREC)
        sums = sums + jax.lax.dot_general(
            ohv, oh_lo, _CONTRACT0, preferred_element_type=jnp.float32,
            precision=_PREC)
        sumsq = sumsq + jax.lax.dot_general(
            ohv * cos_f, oh_lo, _CONTRACT0, preferred_element_type=jnp.float32,
            precision=_PREC)
        cosf_ref[pl.ds(j * _BS, _BS), :] = cos_f
        label_ref[pl.ds(j * _BS, _BS), :] = lab.astype(jnp.float32)
        return counts, sums, sumsq

    zero_mat = jnp.zeros((_HI, _HI), jnp.float32)
    counts, sums, sumsq = jax.lax.fori_loop(
        0, _NB, phase1, (zero_mat, zero_mat, zero_mat))
    mean = sums / jnp.maximum(counts, 1.0)
    sq = sumsq - counts * mean * mean
    var = sq / jnp.maximum(counts - 1.0, 1.0)
    inv_std = jax.lax.rsqrt(jnp.maximum(var, 1e-12))
    stats = jnp.concatenate([mean, inv_std], axis=1)

    def phase2(j, _):
        cos_f = cosf_ref[pl.ds(j * _BS, _BS), :]
        lab = label_ref[pl.ds(j * _BS, _BS), :].astype(jnp.int32)
        oh_hi, oh_lo = onehots(lab)
        rows = jnp.dot(oh_hi, stats, preferred_element_type=jnp.float32,
                       precision=_PREC)
        mean_g = jnp.sum(rows[:, :_HI] * oh_lo, axis=1, keepdims=True)
        isd_g = jnp.sum(rows[:, _HI:] * oh_lo, axis=1, keepdims=True)
        z = (cos_f - mean_g) * isd_g
        pdf = jnp.exp(-0.5 * z * z) * isd_g * _INV_SQRT_2PI
        w = jnp.where(cos_f < mean_g, pdf, 1.0)
        weight_ref[pl.ds(j * _BS, _BS), :] = w
        return 0

    jax.lax.fori_loop(0, _NB, phase2, 0)


@functools.partial(jax.jit, static_argnames=())
def kernel(feature, pred, unlabeled_index, centroids):
    del pred, unlabeled_index
    label2d, weight2d = pl.pallas_call(
        _cluster_kernel,
        out_shape=(
            jax.ShapeDtypeStruct((_B, 1), jnp.float32),
            jax.ShapeDtypeStruct((_B, 1), jnp.float32),
        ),
        scratch_shapes=[
            pltpu.VMEM((_B, 1), jnp.float32),
        ],
    )(feature, jnp.zeros((_D, _KP), jnp.float32).at[:, :_K].set(centroids.T))
    return label2d.reshape(_B), weight2d.reshape(_B)


# speedup vs baseline: 1.3074x; 1.2625x over previous
import functools

import jax
import jax.numpy as jnp
from jax.experimental import pallas as pl
from jax.experimental.pallas import tpu as pltpu

_B = 16384
_D = 64
_K = 1000
_BS = 2048
_NB = _B // _BS
_HI = 32

_INV_SQRT_2PI = 0.3989422804014327

_CONTRACT0 = (((0,), (0,)), ((), ()))
_CONTRACT_NT = (((1,), (1,)), ((), ()))
_PREC = jax.lax.Precision.HIGHEST


def _cluster_kernel(feature_ref, cnt_ref, label_ref, weight_ref, cosf_ref):
    eps = 1e-8
    cnt = cnt_ref[...]
    cnorm = jnp.sqrt(jnp.sum(cnt * cnt, axis=1, keepdims=True))
    cnn = cnt / jnp.maximum(cnorm, eps)

    col_iota = jax.lax.broadcasted_iota(jnp.int32, (_BS, _K), 1)
    iota32 = jax.lax.broadcasted_iota(jnp.int32, (_BS, _HI), 1)

    def onehots(lab):
        oh_hi = (iota32 == (lab >> 5)).astype(jnp.float32)
        oh_lo = (iota32 == (lab & 31)).astype(jnp.float32)
        return oh_hi, oh_lo

    def phase1(j, carry):
        counts, sums, sumsq = carry
        f = feature_ref[pl.ds(j * _BS, _BS), :]
        fnorm = jnp.sqrt(jnp.sum(f * f, axis=1, keepdims=True))
        fn = f / jnp.maximum(fnorm, eps)
        cos = jax.lax.dot_general(
            fn, cnn, _CONTRACT_NT,
            preferred_element_type=jnp.float32)
        cos_f = jnp.max(cos, axis=1, keepdims=True)
        lab = jnp.min(jnp.where(cos == cos_f, col_iota, _K), axis=1,
                      keepdims=True)
        oh_hi, oh_lo = onehots(lab)
        ohv = oh_hi * cos_f
        counts = counts + jax.lax.dot_general(
            oh_hi, oh_lo, _CONTRACT0, preferred_element_type=jnp.float32,
            precision=_PREC)
        sums = sums + jax.lax.dot_general(
            ohv, oh_lo, _CONTRACT0, preferred_element_type=jnp.float32,
            precision=_PREC)
        sumsq = sumsq + jax.lax.dot_general(
            ohv * cos_f, oh_lo, _CONTRACT0, preferred_element_type=jnp.float32,
            precision=_PREC)
        cosf_ref[pl.ds(j * _BS, _BS), :] = cos_f
        label_ref[pl.ds(j * _BS, _BS), :] = lab.astype(jnp.float32)
        return counts, sums, sumsq

    zero_mat = jnp.zeros((_HI, _HI), jnp.float32)
    counts, sums, sumsq = jax.lax.fori_loop(
        0, _NB, phase1, (zero_mat, zero_mat, zero_mat))
    mean = sums / jnp.maximum(counts, 1.0)
    sq = sumsq - counts * mean * mean
    var = sq / jnp.maximum(counts - 1.0, 1.0)
    inv_std = jax.lax.rsqrt(jnp.maximum(var, 1e-12))
    stats = jnp.concatenate([mean, inv_std], axis=1)

    def phase2(j, _):
        cos_f = cosf_ref[pl.ds(j * _BS, _BS), :]
        lab = label_ref[pl.ds(j * _BS, _BS), :].astype(jnp.int32)
        oh_hi, oh_lo = onehots(lab)
        rows = jnp.dot(oh_hi, stats, preferred_element_type=jnp.float32,
                       precision=_PREC)
        mean_g = jnp.sum(rows[:, :_HI] * oh_lo, axis=1, keepdims=True)
        isd_g = jnp.sum(rows[:, _HI:] * oh_lo, axis=1, keepdims=True)
        z = (cos_f - mean_g) * isd_g
        pdf = jnp.exp(-0.5 * z * z) * isd_g * _INV_SQRT_2PI
        w = jnp.where(cos_f < mean_g, pdf, 1.0)
        weight_ref[pl.ds(j * _BS, _BS), :] = w
        return 0

    jax.lax.fori_loop(0, _NB, phase2, 0)


@functools.partial(jax.jit, static_argnames=())
def kernel(feature, pred, unlabeled_index, centroids):
    del pred, unlabeled_index
    label2d, weight2d = pl.pallas_call(
        _cluster_kernel,
        out_shape=(
            jax.ShapeDtypeStruct((_B, 1), jnp.float32),
            jax.ShapeDtypeStruct((_B, 1), jnp.float32),
        ),
        scratch_shapes=[
            pltpu.VMEM((_B, 1), jnp.float32),
        ],
    )(feature, centroids)
    return label2d.reshape(_B), weight2d.reshape(_B)
